# Initial kernel scaffold; baseline (speedup 1.0000x reference)
import jax
import jax.numpy as jnp
from jax.experimental import pallas as pl


def kernel(xyz):
    raise NotImplementedError("write your pallas kernel here")


# speedup vs baseline: 1.5023x; 1.5023x over previous
import jax
import jax.numpy as jnp
from jax.experimental import pallas as pl
from jax.experimental.pallas import tpu as pltpu

B, N, G, K = 16, 8192, 512, 32


def _fps_body(x_ref, y_ref, z_ref, cidx_ref, cx_ref, cy_ref, cz_ref):
    X = x_ref[...]
    Y = y_ref[...]
    Z = z_ref[...]
    lane_n = jax.lax.broadcasted_iota(jnp.int32, (B, N), 1)
    lane_g = jax.lax.broadcasted_iota(jnp.int32, (B, G), 1)
    cidx_ref[...] = jnp.zeros((B, G), jnp.int32)
    cx_ref[...] = jnp.zeros((B, G), jnp.float32)
    cy_ref[...] = jnp.zeros((B, G), jnp.float32)
    cz_ref[...] = jnp.zeros((B, G), jnp.float32)

    def step(t, carry):
        D, far = carry
        sel = lane_n == far
        cx = jnp.sum(jnp.where(sel, X, 0.0), axis=1, keepdims=True)
        cy = jnp.sum(jnp.where(sel, Y, 0.0), axis=1, keepdims=True)
        cz = jnp.sum(jnp.where(sel, Z, 0.0), axis=1, keepdims=True)
        onehot = lane_g == t
        cidx_ref[...] = cidx_ref[...] + jnp.where(onehot, far, 0)
        cx_ref[...] = cx_ref[...] + jnp.where(onehot, cx, 0.0)
        cy_ref[...] = cy_ref[...] + jnp.where(onehot, cy, 0.0)
        cz_ref[...] = cz_ref[...] + jnp.where(onehot, cz, 0.0)
        dx = X - cx
        dy = Y - cy
        dz = Z - cz
        d = dx * dx + dy * dy + dz * dz
        D = jnp.minimum(D, d)
        m = jnp.max(D, axis=1, keepdims=True)
        cand = jnp.where(D == m, lane_n, N)
        far = jnp.min(cand, axis=1, keepdims=True).astype(jnp.int32)
        return D, far

    init = (
        jnp.full((B, N), 1e10, jnp.float32),
        jnp.zeros((B, 1), jnp.int32),
    )
    jax.lax.fori_loop(0, G, step, init)


def _fps(X, Y, Z):
    return pl.pallas_call(
        _fps_body,
        out_shape=[
            jax.ShapeDtypeStruct((B, G), jnp.int32),
            jax.ShapeDtypeStruct((B, G), jnp.float32),
            jax.ShapeDtypeStruct((B, G), jnp.float32),
            jax.ShapeDtypeStruct((B, G), jnp.float32),
        ],
    )(X, Y, Z)


def kernel(xyz):
    planes = jnp.transpose(xyz, (2, 0, 1))
    X, Y, Z = planes[0], planes[1], planes[2]
    cidx, cxs, cys, czs = _fps(X, Y, Z)
    center = jnp.stack([cxs, cys, czs], axis=-1)

    diff = center[:, :, None, :] - xyz[:, None, :, :]
    dist = jnp.sqrt(jnp.sum(diff * diff, axis=-1))
    _, idx = jax.lax.top_k(-dist, K)
    batch = jnp.arange(B).reshape(B, 1, 1)
    neighborhood = xyz[batch, idx] - center[:, :, None, :]
    return (neighborhood, center, idx)


# speedup vs baseline: 11.1525x; 7.4234x over previous
import jax
import jax.numpy as jnp
from jax import lax
from jax.experimental import pallas as pl
from jax.experimental.pallas import tpu as pltpu
from jax.experimental.pallas import tpu_sc as plsc

B, N, G, K = 16, 8192, 512, 32
L = 16
NW = 32
GPW = (B * G) // NW
CAP = 288
CAPV = CAP // L
THRESH = 160
NV = N // L


def _fps_body(x_ref, y_ref, z_ref, cidx_ref, cx_ref, cy_ref, cz_ref):
    X = x_ref[...]
    Y = y_ref[...]
    Z = z_ref[...]
    lane_n = jax.lax.broadcasted_iota(jnp.int32, (B, N), 1)
    lane_g = jax.lax.broadcasted_iota(jnp.int32, (B, G), 1)
    cidx_ref[...] = jnp.zeros((B, G), jnp.int32)
    cx_ref[...] = jnp.zeros((B, G), jnp.float32)
    cy_ref[...] = jnp.zeros((B, G), jnp.float32)
    cz_ref[...] = jnp.zeros((B, G), jnp.float32)

    def step(t, carry):
        D, far = carry
        sel = lane_n == far
        cx = jnp.sum(jnp.where(sel, X, 0.0), axis=1, keepdims=True)
        cy = jnp.sum(jnp.where(sel, Y, 0.0), axis=1, keepdims=True)
        cz = jnp.sum(jnp.where(sel, Z, 0.0), axis=1, keepdims=True)
        onehot = lane_g == t
        cidx_ref[...] = cidx_ref[...] + jnp.where(onehot, far, 0)
        cx_ref[...] = cx_ref[...] + jnp.where(onehot, cx, 0.0)
        cy_ref[...] = cy_ref[...] + jnp.where(onehot, cy, 0.0)
        cz_ref[...] = cz_ref[...] + jnp.where(onehot, cz, 0.0)
        dx = X - cx
        dy = Y - cy
        dz = Z - cz
        d = dx * dx + dy * dy + dz * dz
        D = jnp.minimum(D, d)
        m = jnp.max(D, axis=1, keepdims=True)
        cand = jnp.where(D == m, lane_n, N)
        far = jnp.min(cand, axis=1, keepdims=True).astype(jnp.int32)
        return D, far

    init = (
        jnp.full((B, N), 1e10, jnp.float32),
        jnp.zeros((B, 1), jnp.int32),
    )
    jax.lax.fori_loop(0, G, step, init)


def _fps(X, Y, Z):
    return pl.pallas_call(
        _fps_body,
        out_shape=[
            jax.ShapeDtypeStruct((B, G), jnp.int32),
            jax.ShapeDtypeStruct((B, G), jnp.float32),
            jax.ShapeDtypeStruct((B, G), jnp.float32),
            jax.ShapeDtypeStruct((B, G), jnp.float32),
        ],
    )(X, Y, Z)


def _splat_last(v):
    return lax.rev(v, (0,))[0]


def _merge2(ak, av, bk, bv):
    brk = lax.rev(bk, (0,))
    brv = lax.rev(bv, (0,))
    mm = ak <= brk
    lk = jnp.where(mm, ak, brk)
    lv = jnp.where(mm, av, brv)
    hk = jnp.where(mm, brk, ak)
    hv = jnp.where(mm, brv, av)
    s0k, s0v = plsc.sort_key_val(lk, lv)
    s1k, s1v = plsc.sort_key_val(hk, hv)
    return s0k, s0v, s1k, s1v


def _top32_of_list(cvals, cidxv, cnt):
    k0 = cvals[0:L]
    v0 = cidxv[0:L]
    k1 = cvals[L:2 * L]
    v1 = cidxv[L:2 * L]
    r0k, r0v = plsc.sort_key_val(k0, v0)
    r1k, r1v = plsc.sort_key_val(k1, v1)
    r0k, r0v, r1k, r1v = _merge2(r0k, r0v, r1k, r1v)

    nvregs = lax.div(cnt + (L - 1), L)

    def mstep(j, R):
        q0k, q0v, q1k, q1v = R
        ck = cvals[pl.ds(j * L, L)]
        cv = cidxv[pl.ds(j * L, L)]
        sk, sv = plsc.sort_key_val(ck, cv)
        srk = lax.rev(sk, (0,))
        srv = lax.rev(sv, (0,))
        mm = q1k <= srk
        lk = jnp.where(mm, q1k, srk)
        lv = jnp.where(mm, q1v, srv)
        ak, av = plsc.sort_key_val(lk, lv)
        return _merge2(q0k, q0v, ak, av)

    return lax.fori_loop(2, nvregs, mstep, (r0k, r0v, r1k, r1v))


def _knn_body(xh, yh, zh, cxh, cyh, czh, idxh, nbh,
              xv, yv, zv, cxv, cyv, czv, cvals, cidxv, io, nbx, nby, nbz):
    c = lax.axis_index("c")
    s = lax.axis_index("s")
    w = s * 2 + c
    b = s
    h = c

    pltpu.sync_copy(xh.at[pl.ds(b * N, N)], xv)
    pltpu.sync_copy(yh.at[pl.ds(b * N, N)], yv)
    pltpu.sync_copy(zh.at[pl.ds(b * N, N)], zv)
    cbase = b * G + h * GPW
    pltpu.sync_copy(cxh.at[pl.ds(cbase, GPW)], cxv.at[pl.ds(0, GPW)])
    pltpu.sync_copy(cyh.at[pl.ds(cbase, GPW)], cyv.at[pl.ds(0, GPW)])
    pltpu.sync_copy(czh.at[pl.ds(cbase, GPW)], czv.at[pl.ds(0, GPW)])

    iota = lax.iota(jnp.int32, L)
    infv = jnp.full((L,), jnp.inf, jnp.float32)

    def center_body(g, _unused):
        cx = cxv[pl.ds(g, L)][0]
        cy = cyv[pl.ds(g, L)][0]
        cz = czv[pl.ds(g, L)][0]

        def initf(j, _):
            cvals[pl.ds(j * L, L)] = infv
            return 0

        lax.fori_loop(0, CAPV, initf, 0)

        def rebuild_inplace(args):
            Tv, cnt = args
            r0k, r0v, r1k, r1v = _top32_of_list(cvals, cidxv, cnt)
            cvals[0:L] = r0k
            cvals[L:2 * L] = r1k
            cidxv[0:L] = r0v
            cidxv[L:2 * L] = r1v

            def refill(j, _):
                cvals[pl.ds(j * L, L)] = infv
                return 0

            lax.fori_loop(2, CAPV, refill, 0)
            Tv = jnp.zeros((L,), jnp.float32) + _splat_last(r1k)
            return Tv, jnp.int32(2 * L)

        def stream_outer(ov, carry):
            Tv, cnt = carry
            for j in range(8):
                base = (ov * 8 + j) * L
                xs = xv[pl.ds(base, L)]
                ys = yv[pl.ds(base, L)]
                zs = zv[pl.ds(base, L)]
                dx = xs - cx
                dy = ys - cy
                dz = zs - cz
                d = dx * dx + dy * dy
                d = d + dz * dz
                m = d < Tv
                pc = plsc.cumsum(m.astype(jnp.int32))
                dest = (pc - 1) + cnt
                plsc.store_scatter(cvals, [dest], d, mask=m)
                plsc.store_scatter(cidxv, [dest], iota + base, mask=m)
                cnt = cnt + _splat_last(pc)
            return lax.cond(cnt >= THRESH, rebuild_inplace,
                            lambda a: a, (Tv, cnt))

        Tv0 = jnp.full((L,), jnp.inf, jnp.float32)
        Tv, cnt = lax.fori_loop(0, NV // 8, stream_outer,
                                (Tv0, jnp.int32(0)))

        r0k, r0v, r1k, r1v = _top32_of_list(cvals, cidxv, cnt)

        for vals, off in ((r0v, 0), (r1v, L)):
            px = plsc.load_gather(xv, [vals])
            py = plsc.load_gather(yv, [vals])
            pz = plsc.load_gather(zv, [vals])
            obase = g * K + off
            nbx[pl.ds(obase, L)] = px - cx
            nby[pl.ds(obase, L)] = py - cy
            nbz[pl.ds(obase, L)] = pz - cz
            io[pl.ds(obase, L)] = vals
        return 0

    lax.fori_loop(0, GPW, center_body, 0)

    BGK = B * G * K
    pltpu.sync_copy(io, idxh.at[pl.ds(w * (GPW * K), GPW * K)])
    pltpu.sync_copy(nbx, nbh.at[pl.ds(0 * BGK + w * (GPW * K), GPW * K)])
    pltpu.sync_copy(nby, nbh.at[pl.ds(1 * BGK + w * (GPW * K), GPW * K)])
    pltpu.sync_copy(nbz, nbh.at[pl.ds(2 * BGK + w * (GPW * K), GPW * K)])


def _sc_knn(X, Y, Z, cxs, cys, czs):
    mesh = plsc.VectorSubcoreMesh(core_axis_name="c", subcore_axis_name="s",
                                  num_cores=2, num_subcores=16)
    return pl.kernel(
        _knn_body,
        out_type=[
            jax.ShapeDtypeStruct((B * G * K,), jnp.int32),
            jax.ShapeDtypeStruct((3 * B * G * K,), jnp.float32),
        ],
        mesh=mesh,
        compiler_params=pltpu.CompilerParams(needs_layout_passes=False),
        scratch_types=[
            pltpu.VMEM((N,), jnp.float32),
            pltpu.VMEM((N,), jnp.float32),
            pltpu.VMEM((N,), jnp.float32),
            pltpu.VMEM((GPW + L,), jnp.float32),
            pltpu.VMEM((GPW + L,), jnp.float32),
            pltpu.VMEM((GPW + L,), jnp.float32),
            pltpu.VMEM((CAP,), jnp.float32),
            pltpu.VMEM((CAP,), jnp.int32),
            pltpu.VMEM((GPW * K,), jnp.int32),
            pltpu.VMEM((GPW * K,), jnp.float32),
            pltpu.VMEM((GPW * K,), jnp.float32),
            pltpu.VMEM((GPW * K,), jnp.float32),
        ],
    )(X, Y, Z, cxs, cys, czs)


def kernel(xyz):
    planes = jnp.transpose(xyz, (2, 0, 1))
    X, Y, Z = planes[0], planes[1], planes[2]
    cidx, cxs, cys, czs = _fps(X, Y, Z)
    center = jnp.stack([cxs, cys, czs], axis=-1)

    idx_flat, nb_planes = _sc_knn(
        X.reshape(-1), Y.reshape(-1), Z.reshape(-1),
        cxs.reshape(-1), cys.reshape(-1), czs.reshape(-1))
    idx = idx_flat.reshape(B, G, K)
    neighborhood = jnp.transpose(nb_planes.reshape(3, B, G, K), (1, 2, 3, 0))
    return (neighborhood, center, idx)


# speedup vs baseline: 11.4097x; 1.0231x over previous
import jax
import jax.numpy as jnp
from jax import lax
from jax.experimental import pallas as pl
from jax.experimental.pallas import tpu as pltpu
from jax.experimental.pallas import tpu_sc as plsc

B, N, G, K = 16, 8192, 512, 32
L = 16
NW = 32
GPW = (B * G) // NW
UNROLL = 16
CAP = 512
CAPV = CAP // L
THRESH = CAP - UNROLL * L
NV = N // L


def _fps_body(x_ref, y_ref, z_ref, cidx_ref, cx_ref, cy_ref, cz_ref):
    X = x_ref[...]
    Y = y_ref[...]
    Z = z_ref[...]
    lane_n = jax.lax.broadcasted_iota(jnp.int32, (B, N), 1)
    lane_g = jax.lax.broadcasted_iota(jnp.int32, (B, G), 1)
    cidx_ref[...] = jnp.zeros((B, G), jnp.int32)
    cx_ref[...] = jnp.zeros((B, G), jnp.float32)
    cy_ref[...] = jnp.zeros((B, G), jnp.float32)
    cz_ref[...] = jnp.zeros((B, G), jnp.float32)

    def step(t, carry):
        D, far = carry
        sel = lane_n == far
        cx = jnp.sum(jnp.where(sel, X, 0.0), axis=1, keepdims=True)
        cy = jnp.sum(jnp.where(sel, Y, 0.0), axis=1, keepdims=True)
        cz = jnp.sum(jnp.where(sel, Z, 0.0), axis=1, keepdims=True)
        onehot = lane_g == t
        cidx_ref[...] = cidx_ref[...] + jnp.where(onehot, far, 0)
        cx_ref[...] = cx_ref[...] + jnp.where(onehot, cx, 0.0)
        cy_ref[...] = cy_ref[...] + jnp.where(onehot, cy, 0.0)
        cz_ref[...] = cz_ref[...] + jnp.where(onehot, cz, 0.0)
        dx = X - cx
        dy = Y - cy
        dz = Z - cz
        d = dx * dx + dy * dy + dz * dz
        D = jnp.minimum(D, d)
        m = jnp.max(D, axis=1, keepdims=True)
        cand = jnp.where(D == m, lane_n, N)
        far = jnp.min(cand, axis=1, keepdims=True).astype(jnp.int32)
        return D, far

    init = (
        jnp.full((B, N), 1e10, jnp.float32),
        jnp.zeros((B, 1), jnp.int32),
    )
    jax.lax.fori_loop(0, G, step, init)


def _fps(X, Y, Z):
    return pl.pallas_call(
        _fps_body,
        out_shape=[
            jax.ShapeDtypeStruct((B, G), jnp.int32),
            jax.ShapeDtypeStruct((B, G), jnp.float32),
            jax.ShapeDtypeStruct((B, G), jnp.float32),
            jax.ShapeDtypeStruct((B, G), jnp.float32),
        ],
    )(X, Y, Z)


def _splat_last(v):
    return lax.rev(v, (0,))[0]


def _merge2(ak, av, bk, bv):
    brk = lax.rev(bk, (0,))
    brv = lax.rev(bv, (0,))
    mm = ak <= brk
    lk = jnp.where(mm, ak, brk)
    lv = jnp.where(mm, av, brv)
    hk = jnp.where(mm, brk, ak)
    hv = jnp.where(mm, brv, av)
    s0k, s0v = plsc.sort_key_val(lk, lv)
    s1k, s1v = plsc.sort_key_val(hk, hv)
    return s0k, s0v, s1k, s1v


def _top32_of_list(cvals, cidxv, cnt):
    k0 = cvals[0:L]
    v0 = cidxv[0:L]
    k1 = cvals[L:2 * L]
    v1 = cidxv[L:2 * L]
    r0k, r0v = plsc.sort_key_val(k0, v0)
    r1k, r1v = plsc.sort_key_val(k1, v1)
    r0k, r0v, r1k, r1v = _merge2(r0k, r0v, r1k, r1v)

    nvregs = lax.div(cnt + (L - 1), L)

    def mstep(j, R):
        q0k, q0v, q1k, q1v = R
        ck = cvals[pl.ds(j * L, L)]
        cv = cidxv[pl.ds(j * L, L)]
        sk, sv = plsc.sort_key_val(ck, cv)
        srk = lax.rev(sk, (0,))
        srv = lax.rev(sv, (0,))
        mm = q1k <= srk
        lk = jnp.where(mm, q1k, srk)
        lv = jnp.where(mm, q1v, srv)
        ak, av = plsc.sort_key_val(lk, lv)
        return _merge2(q0k, q0v, ak, av)

    return lax.fori_loop(2, nvregs, mstep, (r0k, r0v, r1k, r1v))


def _knn_body(xh, yh, zh, cxh, cyh, czh, idxh, nbh,
              xv, yv, zv, cxv, cyv, czv, cvals, cidxv, io, nbx, nby, nbz):
    c = lax.axis_index("c")
    s = lax.axis_index("s")
    w = s * 2 + c
    b = s
    h = c

    pltpu.sync_copy(xh.at[pl.ds(b * N, N)], xv)
    pltpu.sync_copy(yh.at[pl.ds(b * N, N)], yv)
    pltpu.sync_copy(zh.at[pl.ds(b * N, N)], zv)
    cbase = b * G + h * GPW
    pltpu.sync_copy(cxh.at[pl.ds(cbase, GPW)], cxv.at[pl.ds(0, GPW)])
    pltpu.sync_copy(cyh.at[pl.ds(cbase, GPW)], cyv.at[pl.ds(0, GPW)])
    pltpu.sync_copy(czh.at[pl.ds(cbase, GPW)], czv.at[pl.ds(0, GPW)])

    iota = lax.iota(jnp.int32, L)
    infv = jnp.full((L,), jnp.inf, jnp.float32)

    def center_body(g, _unused):
        cx = cxv[pl.ds(g, L)][0]
        cy = cyv[pl.ds(g, L)][0]
        cz = czv[pl.ds(g, L)][0]

        for j in range(CAPV):
            cvals[j * L:(j + 1) * L] = infv

        def rebuild_inplace(args):
            Tv, cntv = args
            cnt = cntv[0]
            r0k, r0v, r1k, r1v = _top32_of_list(cvals, cidxv, cnt)
            cvals[0:L] = r0k
            cvals[L:2 * L] = r1k
            cidxv[0:L] = r0v
            cidxv[L:2 * L] = r1v

            def refill(j, _):
                cvals[pl.ds(j * L, L)] = infv
                return 0

            lax.fori_loop(2, CAPV, refill, 0)
            Tv = jnp.zeros((L,), jnp.float32) + _splat_last(r1k)
            return Tv, jnp.full((L,), 2 * L, jnp.int32)

        def stream_outer(ov, carry):
            Tv, cntv = carry
            for j in range(UNROLL):
                base = (ov * UNROLL + j) * L
                xs = xv[pl.ds(base, L)]
                ys = yv[pl.ds(base, L)]
                zs = zv[pl.ds(base, L)]
                dx = xs - cx
                dy = ys - cy
                dz = zs - cz
                d = dx * dx + dy * dy
                d = d + dz * dz
                m = d < Tv
                pc = plsc.cumsum(m.astype(jnp.int32))
                dest = (pc - 1) + cntv
                plsc.store_scatter(cvals, [dest], d, mask=m)
                plsc.store_scatter(cidxv, [dest], iota + base, mask=m)
                cntv = cntv + plsc.all_reduce_population_count(m)
            return lax.cond(cntv[0] >= THRESH, rebuild_inplace,
                            lambda a: a, (Tv, cntv))

        Tv0 = jnp.full((L,), jnp.inf, jnp.float32)
        cnt0 = jnp.zeros((L,), jnp.int32)
        Tv, cntv = lax.fori_loop(0, NV // UNROLL, stream_outer,
                                 (Tv0, cnt0))

        r0k, r0v, r1k, r1v = _top32_of_list(cvals, cidxv, cntv[0])

        for vals, off in ((r0v, 0), (r1v, L)):
            px = plsc.load_gather(xv, [vals])
            py = plsc.load_gather(yv, [vals])
            pz = plsc.load_gather(zv, [vals])
            obase = g * K + off
            nbx[pl.ds(obase, L)] = px - cx
            nby[pl.ds(obase, L)] = py - cy
            nbz[pl.ds(obase, L)] = pz - cz
            io[pl.ds(obase, L)] = vals
        return 0

    lax.fori_loop(0, GPW, center_body, 0)

    BGK = B * G * K
    pltpu.sync_copy(io, idxh.at[pl.ds(w * (GPW * K), GPW * K)])
    pltpu.sync_copy(nbx, nbh.at[pl.ds(0 * BGK + w * (GPW * K), GPW * K)])
    pltpu.sync_copy(nby, nbh.at[pl.ds(1 * BGK + w * (GPW * K), GPW * K)])
    pltpu.sync_copy(nbz, nbh.at[pl.ds(2 * BGK + w * (GPW * K), GPW * K)])


def _sc_knn(X, Y, Z, cxs, cys, czs):
    mesh = plsc.VectorSubcoreMesh(core_axis_name="c", subcore_axis_name="s",
                                  num_cores=2, num_subcores=16)
    return pl.kernel(
        _knn_body,
        out_type=[
            jax.ShapeDtypeStruct((B * G * K,), jnp.int32),
            jax.ShapeDtypeStruct((3 * B * G * K,), jnp.float32),
        ],
        mesh=mesh,
        compiler_params=pltpu.CompilerParams(needs_layout_passes=False),
        scratch_types=[
            pltpu.VMEM((N,), jnp.float32),
            pltpu.VMEM((N,), jnp.float32),
            pltpu.VMEM((N,), jnp.float32),
            pltpu.VMEM((GPW + L,), jnp.float32),
            pltpu.VMEM((GPW + L,), jnp.float32),
            pltpu.VMEM((GPW + L,), jnp.float32),
            pltpu.VMEM((CAP,), jnp.float32),
            pltpu.VMEM((CAP,), jnp.int32),
            pltpu.VMEM((GPW * K,), jnp.int32),
            pltpu.VMEM((GPW * K,), jnp.float32),
            pltpu.VMEM((GPW * K,), jnp.float32),
            pltpu.VMEM((GPW * K,), jnp.float32),
        ],
    )(X, Y, Z, cxs, cys, czs)


def kernel(xyz):
    planes = jnp.transpose(xyz, (2, 0, 1))
    X, Y, Z = planes[0], planes[1], planes[2]
    cidx, cxs, cys, czs = _fps(X, Y, Z)
    center = jnp.stack([cxs, cys, czs], axis=-1)

    idx_flat, nb_planes = _sc_knn(
        X.reshape(-1), Y.reshape(-1), Z.reshape(-1),
        cxs.reshape(-1), cys.reshape(-1), czs.reshape(-1))
    idx = idx_flat.reshape(B, G, K)
    neighborhood = jnp.transpose(nb_planes.reshape(3, B, G, K), (1, 2, 3, 0))
    return (neighborhood, center, idx)


# speedup vs baseline: 28.2259x; 2.4738x over previous
import jax
import jax.numpy as jnp
from jax import lax
from jax.experimental import pallas as pl
from jax.experimental.pallas import tpu as pltpu
from jax.experimental.pallas import tpu_sc as plsc

B, N, G, K = 16, 8192, 512, 32
L = 16
NW = 32
GPW = (B * G) // NW
UNROLL = 16
CAP = 512
CAPV = CAP // L
THRESH = CAP - UNROLL * L
NV = N // L


def _fps_body(x_ref, y_ref, z_ref, cidx_ref, cx_ref, cy_ref, cz_ref):
    X = x_ref[...]
    Y = y_ref[...]
    Z = z_ref[...]
    lane_n = jax.lax.broadcasted_iota(jnp.int32, (B, N), 1)
    lane_g = jax.lax.broadcasted_iota(jnp.int32, (B, G), 1)
    cidx_ref[...] = jnp.zeros((B, G), jnp.int32)
    cx_ref[...] = jnp.zeros((B, G), jnp.float32)
    cy_ref[...] = jnp.zeros((B, G), jnp.float32)
    cz_ref[...] = jnp.zeros((B, G), jnp.float32)

    def step(t, carry):
        D, far = carry
        sel = lane_n == far
        cx = jnp.sum(jnp.where(sel, X, 0.0), axis=1, keepdims=True)
        cy = jnp.sum(jnp.where(sel, Y, 0.0), axis=1, keepdims=True)
        cz = jnp.sum(jnp.where(sel, Z, 0.0), axis=1, keepdims=True)
        onehot = lane_g == t
        cidx_ref[...] = cidx_ref[...] + jnp.where(onehot, far, 0)
        cx_ref[...] = cx_ref[...] + jnp.where(onehot, cx, 0.0)
        cy_ref[...] = cy_ref[...] + jnp.where(onehot, cy, 0.0)
        cz_ref[...] = cz_ref[...] + jnp.where(onehot, cz, 0.0)
        dx = X - cx
        dy = Y - cy
        dz = Z - cz
        d = dx * dx + dy * dy + dz * dz
        D = jnp.minimum(D, d)
        m = jnp.max(D, axis=1, keepdims=True)
        cand = jnp.where(D == m, lane_n, N)
        far = jnp.min(cand, axis=1, keepdims=True).astype(jnp.int32)
        return D, far

    init = (
        jnp.full((B, N), 1e10, jnp.float32),
        jnp.zeros((B, 1), jnp.int32),
    )
    jax.lax.fori_loop(0, G, step, init)


def _fps(X, Y, Z):
    return pl.pallas_call(
        _fps_body,
        out_shape=[
            jax.ShapeDtypeStruct((B, G), jnp.int32),
            jax.ShapeDtypeStruct((B, G), jnp.float32),
            jax.ShapeDtypeStruct((B, G), jnp.float32),
            jax.ShapeDtypeStruct((B, G), jnp.float32),
        ],
    )(X, Y, Z)


def _splat_last(v):
    return lax.rev(v, (0,))[0]


def _merge2(ak, av, bk, bv):
    brk = lax.rev(bk, (0,))
    brv = lax.rev(bv, (0,))
    mm = ak <= brk
    lk = jnp.where(mm, ak, brk)
    lv = jnp.where(mm, av, brv)
    hk = jnp.where(mm, brk, ak)
    hv = jnp.where(mm, brv, av)
    s0k, s0v = plsc.sort_key_val(lk, lv)
    s1k, s1v = plsc.sort_key_val(hk, hv)
    return s0k, s0v, s1k, s1v


def _top32_of_list(cvals, cidxv, cnt):
    k0 = cvals[0:L]
    v0 = cidxv[0:L]
    k1 = cvals[L:2 * L]
    v1 = cidxv[L:2 * L]
    r0k, r0v = plsc.sort_key_val(k0, v0)
    r1k, r1v = plsc.sort_key_val(k1, v1)
    r0k, r0v, r1k, r1v = _merge2(r0k, r0v, r1k, r1v)

    nvregs = lax.div(cnt + (L - 1), L)

    def mstep(j, R):
        q0k, q0v, q1k, q1v = R
        ck = cvals[pl.ds(j * L, L)]
        cv = cidxv[pl.ds(j * L, L)]
        sk, sv = plsc.sort_key_val(ck, cv)
        srk = lax.rev(sk, (0,))
        srv = lax.rev(sv, (0,))
        mm = q1k <= srk
        lk = jnp.where(mm, q1k, srk)
        lv = jnp.where(mm, q1v, srv)
        ak, av = plsc.sort_key_val(lk, lv)
        return _merge2(q0k, q0v, ak, av)

    return lax.fori_loop(2, nvregs, mstep, (r0k, r0v, r1k, r1v))


def _knn_body(xh, yh, zh, cxh, cyh, czh, idxh, nbh,
              xv, yv, zv, cxv, cyv, czv, cvals, cidxv, io, nbx, nby, nbz):
    c = lax.axis_index("c")
    s = lax.axis_index("s")
    w = s * 2 + c
    b = s
    h = c

    pltpu.sync_copy(xh.at[pl.ds(b * N, N)], xv)
    pltpu.sync_copy(yh.at[pl.ds(b * N, N)], yv)
    pltpu.sync_copy(zh.at[pl.ds(b * N, N)], zv)
    cbase = b * G + h * GPW
    pltpu.sync_copy(cxh.at[pl.ds(cbase, GPW)], cxv.at[pl.ds(0, GPW)])
    pltpu.sync_copy(cyh.at[pl.ds(cbase, GPW)], cyv.at[pl.ds(0, GPW)])
    pltpu.sync_copy(czh.at[pl.ds(cbase, GPW)], czv.at[pl.ds(0, GPW)])

    iota = lax.iota(jnp.int32, L)
    infv = jnp.full((L,), jnp.inf, jnp.float32)

    def center_body(g, _unused):
        cx = cxv[pl.ds(g, L)][0]
        cy = cyv[pl.ds(g, L)][0]
        cz = czv[pl.ds(g, L)][0]

        for j in range(CAPV):
            cvals[j * L:(j + 1) * L] = infv

        def rebuild_inplace(args):
            Tv, cntv = args
            cnt = cntv[0]
            r0k, r0v, r1k, r1v = _top32_of_list(cvals, cidxv, cnt)
            cvals[0:L] = r0k
            cvals[L:2 * L] = r1k
            cidxv[0:L] = r0v
            cidxv[L:2 * L] = r1v

            def refill(j, _):
                cvals[pl.ds(j * L, L)] = infv
                return 0

            lax.fori_loop(2, CAPV, refill, 0)
            Tv = jnp.zeros((L,), jnp.float32) + _splat_last(r1k)
            return Tv, jnp.full((L,), 2 * L, jnp.int32)

        def stream_outer(ov, carry):
            Tv, cntv = carry
            ds = []
            ms = []
            for j in range(UNROLL):
                base = (ov * UNROLL + j) * L
                xs = xv[pl.ds(base, L)]
                ys = yv[pl.ds(base, L)]
                zs = zv[pl.ds(base, L)]
                dx = xs - cx
                dy = ys - cy
                dz = zs - cz
                d = dx * dx + dy * dy
                d = d + dz * dz
                ds.append(d)
                ms.append(d < Tv)
            pcs = [plsc.cumsum(m.astype(jnp.int32)) for m in ms]
            pcnts = [plsc.all_reduce_population_count(m) for m in ms]
            for j in range(UNROLL):
                base = (ov * UNROLL + j) * L
                dest = (pcs[j] - 1) + cntv
                plsc.store_scatter(cvals, [dest], ds[j], mask=ms[j])
                plsc.store_scatter(cidxv, [dest], iota + base, mask=ms[j])
                cntv = cntv + pcnts[j]
            return lax.cond(cntv[0] >= THRESH, rebuild_inplace,
                            lambda a: a, (Tv, cntv))

        Tv0 = jnp.full((L,), jnp.inf, jnp.float32)
        cnt0 = jnp.zeros((L,), jnp.int32)
        Tv, cntv = lax.fori_loop(0, NV // UNROLL, stream_outer,
                                 (Tv0, cnt0))

        r0k, r0v, r1k, r1v = _top32_of_list(cvals, cidxv, cntv[0])

        for vals, off in ((r0v, 0), (r1v, L)):
            px = plsc.load_gather(xv, [vals])
            py = plsc.load_gather(yv, [vals])
            pz = plsc.load_gather(zv, [vals])
            obase = g * K + off
            nbx[pl.ds(obase, L)] = px - cx
            nby[pl.ds(obase, L)] = py - cy
            nbz[pl.ds(obase, L)] = pz - cz
            io[pl.ds(obase, L)] = vals
        return 0

    lax.fori_loop(0, GPW, center_body, 0)

    BGK = B * G * K
    pltpu.sync_copy(io, idxh.at[pl.ds(w * (GPW * K), GPW * K)])
    pltpu.sync_copy(nbx, nbh.at[pl.ds(0 * BGK + w * (GPW * K), GPW * K)])
    pltpu.sync_copy(nby, nbh.at[pl.ds(1 * BGK + w * (GPW * K), GPW * K)])
    pltpu.sync_copy(nbz, nbh.at[pl.ds(2 * BGK + w * (GPW * K), GPW * K)])


def _sc_knn(X, Y, Z, cxs, cys, czs):
    mesh = plsc.VectorSubcoreMesh(core_axis_name="c", subcore_axis_name="s",
                                  num_cores=2, num_subcores=16)
    return pl.kernel(
        _knn_body,
        out_type=[
            jax.ShapeDtypeStruct((B * G * K,), jnp.int32),
            jax.ShapeDtypeStruct((3 * B * G * K,), jnp.float32),
        ],
        mesh=mesh,
        compiler_params=pltpu.CompilerParams(needs_layout_passes=False),
        scratch_types=[
            pltpu.VMEM((N,), jnp.float32),
            pltpu.VMEM((N,), jnp.float32),
            pltpu.VMEM((N,), jnp.float32),
            pltpu.VMEM((GPW + L,), jnp.float32),
            pltpu.VMEM((GPW + L,), jnp.float32),
            pltpu.VMEM((GPW + L,), jnp.float32),
            pltpu.VMEM((CAP,), jnp.float32),
            pltpu.VMEM((CAP,), jnp.int32),
            pltpu.VMEM((GPW * K,), jnp.int32),
            pltpu.VMEM((GPW * K,), jnp.float32),
            pltpu.VMEM((GPW * K,), jnp.float32),
            pltpu.VMEM((GPW * K,), jnp.float32),
        ],
    )(X, Y, Z, cxs, cys, czs)


def kernel(xyz):
    planes = jnp.transpose(xyz, (2, 0, 1))
    X, Y, Z = planes[0], planes[1], planes[2]
    cidx, cxs, cys, czs = _fps(X, Y, Z)
    center = jnp.stack([cxs, cys, czs], axis=-1)

    idx_flat, nb_planes = _sc_knn(
        X.reshape(-1), Y.reshape(-1), Z.reshape(-1),
        cxs.reshape(-1), cys.reshape(-1), czs.reshape(-1))
    idx = idx_flat.reshape(B, G, K)
    neighborhood = jnp.transpose(nb_planes.reshape(3, B, G, K), (1, 2, 3, 0))
    return (neighborhood, center, idx)
